# Initial kernel scaffold; baseline (speedup 1.0000x reference)
import functools

import jax
import jax.numpy as jnp
from jax import lax
from jax.experimental import pallas as pl
from jax.experimental.pallas import tpu as pltpu
from jax.experimental.pallas import tpu_sc as plsc

N = 10000
D = 128
E = 320000
CHUNK = 128
NUM_CHUNKS = E // CHUNK
NSUB = 16
ROWS_PER_SUB = N // NSUB
MAX_CHUNKS_PER_SUB = -(-NUM_CHUNKS // NSUB)


def _dense_body(mean_ref, var_ref, wmT_ref, bm_ref, wvT_ref, bv_ref,
                m_ref, v_ref):
    m = jnp.dot(mean_ref[...], wmT_ref[...],
                preferred_element_type=jnp.float32) + bm_ref[...]
    v = jnp.dot(var_ref[...], wvT_ref[...],
                preferred_element_type=jnp.float32) + bv_ref[...]
    m = jnp.where(m > 0.0, m, jnp.expm1(m))
    v = jnp.maximum(v, 0.0)
    att = jnp.exp(-v)
    m_ref[...] = m * att
    v_ref[...] = v * att * att


def _dense(mean, var, wmT, bm, wvT, bv):
    return pl.pallas_call(
        _dense_body,
        out_shape=(jax.ShapeDtypeStruct((N, D), jnp.float32),
                   jax.ShapeDtypeStruct((N, D), jnp.float32)),
    )(mean, var, wmT, bm, wvT, bv)


def _sc_spmm(m_s, v_s, row, col, adj0, adj1):
    mesh = plsc.VectorSubcoreMesh(core_axis_name="c", subcore_axis_name="s")

    @functools.partial(
        pl.kernel,
        out_type=(jax.ShapeDtypeStruct((N, D), jnp.float32),
                  jax.ShapeDtypeStruct((N, D), jnp.float32)),
        mesh=mesh,
        scratch_types=[
            pltpu.VMEM_SHARED((N, D), jnp.float32),
            pltpu.VMEM((CHUNK,), jnp.int32),
            pltpu.VMEM((CHUNK,), jnp.int32),
            pltpu.VMEM((CHUNK,), jnp.float32),
            pltpu.VMEM((CHUNK, D), jnp.float32),
            pltpu.VMEM((CHUNK, D), jnp.float32),
            pltpu.SemaphoreType.DMA,
        ],
    )
    def sc_kernel(ms_hbm, vs_hbm, row_hbm, col_hbm, a0_hbm, a1_hbm,
                  mo_hbm, vo_hbm, acc, colv, rowv, valv, rowsv, zerov, sem):
        cid = lax.axis_index("c")
        sid = lax.axis_index("s")
        base = sid * ROWS_PER_SUB

        z16 = jnp.zeros((16,), jnp.float32)

        @pl.loop(0, CHUNK)
        def _(i):
            for j in range(8):
                zerov[i, pl.ds(j * 16, 16)] = z16

        @pl.loop(0, 4)
        def _(t):
            pltpu.sync_copy(zerov, acc.at[pl.ds(base + t * CHUNK, CHUNK)])
        pltpu.sync_copy(zerov.at[pl.ds(0, ROWS_PER_SUB - 4 * CHUNK)],
                        acc.at[pl.ds(base + 4 * CHUNK, ROWS_PER_SUB - 4 * CHUNK)])
        plsc.subcore_barrier()

        def process(x_hbm, val_hbm, out_hbm):
            @pl.loop(0, MAX_CHUNKS_PER_SUB)
            def _(t):
                c = sid + t * NSUB

                @pl.when(c < NUM_CHUNKS)
                def _():
                    off = c * CHUNK
                    pltpu.sync_copy(col_hbm.at[pl.ds(off, CHUNK)], colv)
                    pltpu.sync_copy(row_hbm.at[pl.ds(off, CHUNK)], rowv)
                    pltpu.sync_copy(val_hbm.at[pl.ds(off, CHUNK)], valv)
                    pltpu.async_copy(x_hbm.at[colv], rowsv, sem).wait()

                    @pl.loop(0, CHUNK)
                    def _(e):
                        val = valv[e]
                        for j in range(8):
                            slc = (e, pl.ds(j * 16, 16))
                            rowsv[slc] = rowsv[slc] * val

                    pltpu.sync_copy(rowsv, acc.at[rowv], add=True)

            plsc.subcore_barrier()

            @pl.loop(0, 4)
            def _(t):
                pltpu.sync_copy(acc.at[pl.ds(base + t * CHUNK, CHUNK)],
                                out_hbm.at[pl.ds(base + t * CHUNK, CHUNK)])
            pltpu.sync_copy(acc.at[pl.ds(base + 4 * CHUNK, ROWS_PER_SUB - 4 * CHUNK)],
                            out_hbm.at[pl.ds(base + 4 * CHUNK, ROWS_PER_SUB - 4 * CHUNK)])

        @pl.when(cid == 0)
        def _():
            process(ms_hbm, a0_hbm, mo_hbm)

        @pl.when(cid == 1)
        def _():
            process(vs_hbm, a1_hbm, vo_hbm)

    return sc_kernel(m_s, v_s, row, col, adj0, adj1)


def kernel(mean, var, edge_index, adj0_values, adj1_values,
           W_mean, b_mean, W_var, b_var):
    m_s, v_s = _dense(mean, var, W_mean.T, b_mean[None, :], W_var.T,
                      b_var[None, :])
    row = edge_index[0]
    col = edge_index[1]
    return _sc_spmm(m_s, v_s, row, col, adj0_values, adj1_values)


# speedup vs baseline: 3.9452x; 3.9452x over previous
import functools

import jax
import jax.numpy as jnp
from jax import lax
from jax.experimental import pallas as pl
from jax.experimental.pallas import tpu as pltpu
from jax.experimental.pallas import tpu_sc as plsc

N = 10000
D = 128
E = 320000
CHUNK = 128
NUM_CHUNKS = E // CHUNK
NSUB = 16
ROWS_PER_SUB = 624
MAX_CHUNKS_PER_SUB = -(-NUM_CHUNKS // NSUB)


def _dense_body(mean_ref, var_ref, wmT_ref, bm_ref, wvT_ref, bv_ref,
                m_ref, v_ref):
    m = jnp.dot(mean_ref[...], wmT_ref[...],
                preferred_element_type=jnp.float32) + bm_ref[...]
    v = jnp.dot(var_ref[...], wvT_ref[...],
                preferred_element_type=jnp.float32) + bv_ref[...]
    m = jnp.where(m > 0.0, m, jnp.exp(jnp.minimum(m, 0.0)) - 1.0)
    v = jnp.maximum(v, 0.0)
    att = jnp.exp(-v)
    m_ref[...] = m * att
    v_ref[...] = v * att * att


def _dense(mean, var, wmT, bm, wvT, bv):
    return pl.pallas_call(
        _dense_body,
        out_shape=(jax.ShapeDtypeStruct((N, D), jnp.float32),
                   jax.ShapeDtypeStruct((N, D), jnp.float32)),
    )(mean, var, wmT, bm, wvT, bv)


def _sc_spmm(m_s, v_s, row, col, adj0, adj1):
    mesh = plsc.VectorSubcoreMesh(core_axis_name="c", subcore_axis_name="s")

    @functools.partial(
        pl.kernel,
        out_type=(jax.ShapeDtypeStruct((N, D), jnp.float32),
                  jax.ShapeDtypeStruct((N, D), jnp.float32)),
        mesh=mesh,
        scratch_types=[
            pltpu.VMEM_SHARED((N, D), jnp.float32),
            pltpu.VMEM((CHUNK,), jnp.int32),
            pltpu.VMEM((CHUNK,), jnp.int32),
            pltpu.VMEM((CHUNK,), jnp.float32),
            pltpu.VMEM((CHUNK, D), jnp.float32),
            pltpu.VMEM((CHUNK, D), jnp.float32),
            pltpu.SemaphoreType.DMA,
        ],
    )
    def sc_kernel(ms_hbm, vs_hbm, row_hbm, col_hbm, a0_hbm, a1_hbm,
                  mo_hbm, vo_hbm, acc, colv, rowv, valv, rowsv, zerov, sem):
        cid = lax.axis_index("c")
        sid = lax.axis_index("s")
        base = sid * ROWS_PER_SUB

        z16 = jnp.zeros((16,), jnp.float32)

        @pl.loop(0, CHUNK)
        def _(i):
            for j in range(8):
                zerov[i, pl.ds(j * 16, 16)] = z16

        @pl.loop(0, 4)
        def _(t):
            pltpu.sync_copy(zerov, acc.at[pl.ds(base + t * CHUNK, CHUNK)])
        pltpu.sync_copy(zerov.at[pl.ds(0, ROWS_PER_SUB - 4 * CHUNK)],
                        acc.at[pl.ds(base + 4 * CHUNK, ROWS_PER_SUB - 4 * CHUNK)])

        @pl.when(sid == NSUB - 1)
        def _():
            pltpu.sync_copy(zerov.at[pl.ds(0, 16)],
                            acc.at[pl.ds(NSUB * ROWS_PER_SUB, 16)])
        plsc.subcore_barrier()

        def process(x_hbm, val_hbm, out_hbm):
            @pl.loop(0, MAX_CHUNKS_PER_SUB)
            def _(t):
                c = sid + t * NSUB

                @pl.when(c < NUM_CHUNKS)
                def _():
                    off = c * CHUNK
                    pltpu.sync_copy(col_hbm.at[pl.ds(off, CHUNK)], colv)
                    pltpu.sync_copy(row_hbm.at[pl.ds(off, CHUNK)], rowv)
                    pltpu.sync_copy(val_hbm.at[pl.ds(off, CHUNK)], valv)
                    pltpu.async_copy(x_hbm.at[colv], rowsv, sem).wait()

                    @pl.loop(0, CHUNK // 16)
                    def _(g):
                        vals16 = valv[pl.ds(g * 16, 16)]
                        for k in range(16):
                            vk = jnp.full((16,), vals16[k])
                            e = g * 16 + k
                            for j in range(8):
                                slc = (e, pl.ds(j * 16, 16))
                                rowsv[slc] = rowsv[slc] * vk

                    pltpu.sync_copy(rowsv, acc.at[rowv], add=True)

            plsc.subcore_barrier()

            @pl.loop(0, 4)
            def _(t):
                pltpu.sync_copy(acc.at[pl.ds(base + t * CHUNK, CHUNK)],
                                out_hbm.at[pl.ds(base + t * CHUNK, CHUNK)])
            pltpu.sync_copy(acc.at[pl.ds(base + 4 * CHUNK, ROWS_PER_SUB - 4 * CHUNK)],
                            out_hbm.at[pl.ds(base + 4 * CHUNK, ROWS_PER_SUB - 4 * CHUNK)])

            @pl.when(sid == NSUB - 1)
            def _():
                pltpu.sync_copy(acc.at[pl.ds(NSUB * ROWS_PER_SUB, 16)],
                                out_hbm.at[pl.ds(NSUB * ROWS_PER_SUB, 16)])

        @pl.when(cid == 0)
        def _():
            process(ms_hbm, a0_hbm, mo_hbm)

        @pl.when(cid == 1)
        def _():
            process(vs_hbm, a1_hbm, vo_hbm)

    return sc_kernel(m_s, v_s, row, col, adj0, adj1)


def kernel(mean, var, edge_index, adj0_values, adj1_values,
           W_mean, b_mean, W_var, b_var):
    m_s, v_s = _dense(mean, var, W_mean.T, b_mean[None, :], W_var.T,
                      b_var[None, :])
    row = edge_index[0]
    col = edge_index[1]
    return _sc_spmm(m_s, v_s, row, col, adj0_values, adj1_values)
